# baseline (speedup 1.0000x reference)
import functools

import jax
import jax.numpy as jnp
from jax import lax
from jax.experimental import pallas as pl
from jax.experimental.pallas import tpu as pltpu
from jax.experimental.pallas import tpu_sc as plsc


def _scan_body(K, BR, ap_ref, w1_ref, seg_ref, minv_ref, rowi_ref):
    i = pl.program_id(0)
    ap = ap_ref[...]
    d2 = jnp.dot(ap * ap, seg_ref[...], preferred_element_type=jnp.float32)
    d2 = d2 + jnp.dot(ap, w1_ref[...], preferred_element_type=jnp.float32)
    loc_min = jnp.min(d2, axis=0, keepdims=True)
    rows = lax.broadcasted_iota(jnp.int32, d2.shape, 0)
    loc_row = jnp.min(jnp.where(d2 == loc_min, rows, K), axis=0,
                      keepdims=True) + i * BR
    prev_v = jnp.where(i == 0, jnp.inf, minv_ref[...])
    prev_r = jnp.where(i == 0, 0, rowi_ref[...])
    better = loc_min < prev_v
    minv_ref[...] = jnp.where(better, loc_min, prev_v)
    rowi_ref[...] = jnp.where(better, loc_row, prev_r)


def _make_merge_gather(Q, d, P, K):
    mesh = plsc.VectorSubcoreMesh(core_axis_name="c", subcore_axis_name="s")

    @functools.partial(
        pl.kernel, mesh=mesh,
        out_type=jax.ShapeDtypeStruct((Q, d), jnp.float32),
        scratch_types=[
            pltpu.VMEM((P * Q,), jnp.float32),
            pltpu.VMEM((P * Q,), jnp.int32),
            pltpu.VMEM((Q, d), jnp.float32),
            pltpu.VMEM((Q,), jnp.int32),
            pltpu.SemaphoreType.DMA,
        ],
    )
    def merge_gather(minv_hbm, rowi_hbm, table_hbm, out_hbm,
                     minv_v, rowi_v, out_v, idx_v, sem):
        wid = lax.axis_index("s") * 2 + lax.axis_index("c")

        @pl.when(wid == 0)
        def _():
            pltpu.sync_copy(minv_hbm, minv_v)
            pltpu.sync_copy(rowi_hbm, rowi_v)
            copies = []
            for qc in range(Q // 16):
                bv = jnp.full((16,), jnp.inf, jnp.float32)
                bg = jnp.zeros((16,), jnp.int32)
                for j in range(P):
                    off = j * Q + qc * 16
                    v = minv_v[pl.ds(off, 16)]
                    g = rowi_v[pl.ds(off, 16)] * P + j
                    better = (v < bv) | ((v == bv) & (g < bg))
                    bv = jnp.where(better, v, bv)
                    bg = jnp.where(better, g, bg)
                idx_v[pl.ds(qc * 16, 16)] = bg
            for qc in range(Q // 16):
                bgv = idx_v[pl.ds(qc * 16, 16)]
                for m in range(16):
                    copies.append(pltpu.async_copy(
                        table_hbm.at[pl.ds(bgv[m], 1)],
                        out_v.at[pl.ds(qc * 16 + m, 1)], sem))
            for c in copies:
                c.wait()
            pltpu.sync_copy(out_v, out_hbm)

    return merge_gather


def kernel(z, all_z):
    b, l, d = z.shape
    Q = b * l
    K = all_z.shape[0]
    P = 128 // d
    R = K // P
    BR = 5000
    steps = R // BR

    zf = jnp.reshape(z, (Q, d))
    ap = jnp.reshape(all_z, (R, P * d))
    eye = jnp.eye(P, dtype=jnp.float32)
    w1 = jnp.einsum("jk,tq->jtkq", eye, -2.0 * zf.T).reshape(P * d, P * Q)
    seg = jnp.einsum("jk,tq->jtkq", eye,
                     jnp.ones((d, Q), jnp.float32)).reshape(P * d, P * Q)

    minv, rowi = pl.pallas_call(
        functools.partial(_scan_body, K, BR),
        grid=(steps,),
        in_specs=[
            pl.BlockSpec((BR, P * d), lambda i: (i, 0)),
            pl.BlockSpec((P * d, P * Q), lambda i: (0, 0)),
            pl.BlockSpec((P * d, P * Q), lambda i: (0, 0)),
        ],
        out_specs=[
            pl.BlockSpec((1, P * Q), lambda i: (0, 0)),
            pl.BlockSpec((1, P * Q), lambda i: (0, 0)),
        ],
        out_shape=[
            jax.ShapeDtypeStruct((1, P * Q), jnp.float32),
            jax.ShapeDtypeStruct((1, P * Q), jnp.int32),
        ],
    )(ap, w1, seg)

    merge_gather = _make_merge_gather(Q, d, P, K)
    nearest = merge_gather(minv.reshape(P * Q), rowi.reshape(P * Q), all_z)
    return jnp.reshape(nearest, (b, l, d))


# speedup vs baseline: 1.3514x; 1.3514x over previous
import functools

import jax
import jax.numpy as jnp
from jax import lax
from jax.experimental import pallas as pl
from jax.experimental.pallas import tpu as pltpu
from jax.experimental.pallas import tpu_sc as plsc

_NBUF = 4


def _scan_body(K, BK, steps, az_ref, zf_ref, minv_ref, rowi_ref, buf, sem):
    i = pl.program_id(0)

    def start(blk, slot):
        pltpu.make_async_copy(az_ref.at[pl.ds(blk * BK, BK), :],
                              buf.at[slot], sem.at[slot]).start()

    @pl.when(i == 0)
    def _():
        for s in range(_NBUF):
            start(s, s)

    slot = i % _NBUF
    pltpu.make_async_copy(az_ref.at[pl.ds(i * BK, BK), :],
                          buf.at[slot], sem.at[slot]).wait()
    ap = buf[slot]
    zf = zf_ref[...]
    dn = (((1,), (1,)), ((), ()))
    s = lax.dot_general(zf, ap, dn,
                        preferred_element_type=jnp.float32)
    ksq = lax.dot_general(jnp.ones((1, 16), jnp.float32), ap * ap, dn,
                          preferred_element_type=jnp.float32)
    d2 = ksq - 2.0 * s
    loc_min = jnp.min(d2, axis=1, keepdims=True)
    cols = lax.broadcasted_iota(jnp.int32, d2.shape, 1)
    loc_row = jnp.min(jnp.where(d2 == loc_min, cols, K), axis=1,
                      keepdims=True) + i * BK
    prev_v = jnp.where(i == 0, jnp.inf, minv_ref[...])
    prev_r = jnp.where(i == 0, 0, rowi_ref[...])
    better = loc_min < prev_v
    minv_ref[...] = jnp.where(better, loc_min, prev_v)
    rowi_ref[...] = jnp.where(better, loc_row, prev_r)

    @pl.when(i + _NBUF < steps)
    def _():
        start(i + _NBUF, slot)


def _make_gather(Q, d):
    mesh = plsc.VectorSubcoreMesh(core_axis_name="c", subcore_axis_name="s")

    @functools.partial(
        pl.kernel, mesh=mesh,
        out_type=jax.ShapeDtypeStruct((Q, d), jnp.float32),
        scratch_types=[
            pltpu.VMEM((Q,), jnp.int32),
            pltpu.VMEM((Q, d), jnp.float32),
            pltpu.SemaphoreType.DMA,
        ],
    )
    def gather(idx_hbm, table_hbm, out_hbm, idx_v, out_v, sem):
        wid = lax.axis_index("s") * 2 + lax.axis_index("c")

        @pl.when(wid == 0)
        def _():
            pltpu.sync_copy(idx_hbm, idx_v)
            copies = []
            for qc in range(Q // 16):
                bgv = idx_v[pl.ds(qc * 16, 16)]
                for m in range(16):
                    copies.append(pltpu.async_copy(
                        table_hbm.at[pl.ds(bgv[m], 1)],
                        out_v.at[pl.ds(qc * 16 + m, 1)], sem))
            for c in copies:
                c.wait()
            pltpu.sync_copy(out_v, out_hbm)

    return gather


def kernel(z, all_z):
    b, l, d = z.shape
    Q = b * l
    K = all_z.shape[0]
    BK = 12500
    steps = K // BK

    zf = jnp.reshape(z, (Q, d))

    minv, rowi = pl.pallas_call(
        functools.partial(_scan_body, K, BK, steps),
        grid=(steps,),
        in_specs=[
            pl.BlockSpec(memory_space=pl.ANY),
            pl.BlockSpec((Q, d), lambda i: (0, 0)),
        ],
        out_specs=[
            pl.BlockSpec((Q, 1), lambda i: (0, 0)),
            pl.BlockSpec((Q, 1), lambda i: (0, 0)),
        ],
        out_shape=[
            jax.ShapeDtypeStruct((Q, 1), jnp.float32),
            jax.ShapeDtypeStruct((Q, 1), jnp.int32),
        ],
        scratch_shapes=[
            pltpu.VMEM((_NBUF, BK, d), jnp.float32),
            pltpu.SemaphoreType.DMA((_NBUF,)),
        ],
    )(all_z, zf)

    gather = _make_gather(Q, d)
    nearest = gather(rowi.reshape(Q), all_z)
    return jnp.reshape(nearest, (b, l, d))
